# Initial kernel scaffold; baseline (speedup 1.0000x reference)
import jax
import jax.numpy as jnp
from jax.experimental import pallas as pl


def kernel(points, W1, b1, gamma1, beta1):
    raise NotImplementedError("write your pallas kernel here")


# speedup vs baseline: 7.8170x; 7.8170x over previous
import functools

import jax
import jax.numpy as jnp
from jax import lax
from jax.experimental import pallas as pl
from jax.experimental.pallas import tpu as pltpu
from jax.experimental.pallas import tpu_sc as plsc

B = 4
C = 64
N = 2048
K = 32
TN = 256
NT = N // TN
EPS = 1e-5
CNT = B * N * K

SC_CORES = 2
SC_SUBCORES = 16
NWORK = SC_CORES * SC_SUBCORES
ROWS_PER_W = CNT // NWORK
CH = 1024


def _knn_body(xf_ref, xt_ref, w_ref, idx_ref, u_ref, v_ref):
    b = pl.program_id(0)
    x = xf_ref[0]
    xt = xt_ref[0]
    sq = jnp.sum(x * x, axis=0, keepdims=True)
    inner = lax.dot_general(xt, x, (((0,), (0,)), ((), ())),
                            preferred_element_type=jnp.float32)
    score = 2.0 * inner - sq
    iota = lax.broadcasted_iota(jnp.int32, (TN, N), 1)
    cols = []
    cur = score
    for _ in range(K):
        m = jnp.max(cur, axis=1, keepdims=True)
        am = jnp.min(jnp.where(cur == m, iota, N), axis=1, keepdims=True)
        cols.append(am)
        cur = jnp.where(iota == am, -jnp.inf, cur)
    idx_ref[0] = jnp.concatenate(cols, axis=1) + b * N

    w = w_ref[...]
    wa = w[:, :C]
    wb = w[:, C:]
    u_ref[0] = lax.dot_general(xt, wa - wb, (((0,), (1,)), ((), ())),
                               preferred_element_type=jnp.float32)
    v_ref[0] = lax.dot_general(xt, wb, (((0,), (1,)), ((), ())),
                               preferred_element_type=jnp.float32)


def _gather_body(idx_hbm, tab_hbm, out_hbm, idx_v, rows_v, sem):
    wid = lax.axis_index("s") * SC_CORES + lax.axis_index("c")
    for j in range(ROWS_PER_W // CH):
        base = wid * ROWS_PER_W + j * CH
        pltpu.sync_copy(idx_hbm.at[pl.ds(base, CH)], idx_v)
        pltpu.async_copy(tab_hbm.at[idx_v], rows_v, sem).wait()
        pltpu.sync_copy(rows_v, out_hbm.at[pl.ds(base, CH)])


def _reduce_body(g_ref, u_ref, b1_ref, vmax_ref, vmin_ref, ps_ref, pss_ref):
    g = g_ref[0]
    u = u_ref[0]
    b1 = b1_ref[...]
    s = g[:, 0:C]
    vmax = s
    vmin = s
    gsum = s
    gssq = s * s
    for k in range(1, K):
        s = g[:, k * C:(k + 1) * C]
        vmax = jnp.maximum(vmax, s)
        vmin = jnp.minimum(vmin, s)
        gsum = gsum + s
        gssq = gssq + s * s
    vmax_ref[0] = vmax
    vmin_ref[0] = vmin
    ub = u + b1
    ps = float(K) * ub + gsum
    pss = float(K) * ub * ub + 2.0 * ub * gsum + gssq
    ps_ref[0] = jnp.sum(ps, axis=0, keepdims=True)
    pss_ref[0] = jnp.sum(pss, axis=0, keepdims=True)


def _out_body(ps_ref, pss_ref, u_ref, vmax_ref, vmin_ref, b1_ref,
              gam_ref, bet_ref, o_ref):
    mean = jnp.sum(ps_ref[:, 0, :], axis=0, keepdims=True) / CNT
    ex2 = jnp.sum(pss_ref[:, 0, :], axis=0, keepdims=True) / CNT
    var = ex2 - mean * mean
    gam = gam_ref[...]
    bet = bet_ref[...]
    b1 = b1_ref[...]
    scale = gam * lax.rsqrt(var + EPS)
    sel = jnp.where(gam >= 0.0, vmax_ref[0], vmin_ref[0])
    h = (u_ref[0] + b1 + sel - mean) * scale + bet
    o_ref[0] = jnp.maximum(h, 0.0).T


def _gather_call():
    return functools.partial(
        pl.kernel,
        mesh=plsc.VectorSubcoreMesh(
            core_axis_name="c", subcore_axis_name="s", num_cores=SC_CORES
        ),
        out_type=jax.ShapeDtypeStruct((CNT, C), jnp.float32),
        compiler_params=pltpu.CompilerParams(use_tc_tiling_on_sc=False),
        scratch_types=[
            pltpu.VMEM((CH,), jnp.int32),
            pltpu.VMEM((CH, C), jnp.float32),
            pltpu.SemaphoreType.DMA,
        ],
    )


def kernel(points, W1, b1, gamma1, beta1):
    idx, uT, vT = pl.pallas_call(
        _knn_body,
        grid=(B, NT),
        in_specs=[
            pl.BlockSpec((1, C, N), lambda b, t: (b, 0, 0)),
            pl.BlockSpec((1, C, TN), lambda b, t: (b, 0, t)),
            pl.BlockSpec((C, 2 * C), lambda b, t: (0, 0)),
        ],
        out_specs=[
            pl.BlockSpec((1, TN, K), lambda b, t: (b, t, 0)),
            pl.BlockSpec((1, TN, C), lambda b, t: (b, t, 0)),
            pl.BlockSpec((1, TN, C), lambda b, t: (b, t, 0)),
        ],
        out_shape=[
            jax.ShapeDtypeStruct((B, N, K), jnp.int32),
            jax.ShapeDtypeStruct((B, N, C), jnp.float32),
            jax.ShapeDtypeStruct((B, N, C), jnp.float32),
        ],
    )(points, points, W1)

    idx_flat = idx.reshape(CNT)
    tab = vT.reshape(B * N, C)
    g = _gather_call()(_gather_body)(idx_flat, tab)
    g2 = g.reshape(B, N, K * C)

    b1r = b1.reshape(1, C)
    gamr = gamma1.reshape(1, C)
    betr = beta1.reshape(1, C)

    vmax, vmin, ps, pss = pl.pallas_call(
        _reduce_body,
        grid=(B, NT),
        in_specs=[
            pl.BlockSpec((1, TN, K * C), lambda b, t: (b, t, 0)),
            pl.BlockSpec((1, TN, C), lambda b, t: (b, t, 0)),
            pl.BlockSpec((1, C), lambda b, t: (0, 0)),
        ],
        out_specs=[
            pl.BlockSpec((1, TN, C), lambda b, t: (b, t, 0)),
            pl.BlockSpec((1, TN, C), lambda b, t: (b, t, 0)),
            pl.BlockSpec((1, 1, C), lambda b, t: (b * NT + t, 0, 0)),
            pl.BlockSpec((1, 1, C), lambda b, t: (b * NT + t, 0, 0)),
        ],
        out_shape=[
            jax.ShapeDtypeStruct((B, N, C), jnp.float32),
            jax.ShapeDtypeStruct((B, N, C), jnp.float32),
            jax.ShapeDtypeStruct((B * NT, 1, C), jnp.float32),
            jax.ShapeDtypeStruct((B * NT, 1, C), jnp.float32),
        ],
    )(g2, uT, b1r)

    out = pl.pallas_call(
        _out_body,
        grid=(B, NT),
        in_specs=[
            pl.BlockSpec((B * NT, 1, C), lambda b, t: (0, 0, 0)),
            pl.BlockSpec((B * NT, 1, C), lambda b, t: (0, 0, 0)),
            pl.BlockSpec((1, TN, C), lambda b, t: (b, t, 0)),
            pl.BlockSpec((1, TN, C), lambda b, t: (b, t, 0)),
            pl.BlockSpec((1, TN, C), lambda b, t: (b, t, 0)),
            pl.BlockSpec((1, C), lambda b, t: (0, 0)),
            pl.BlockSpec((1, C), lambda b, t: (0, 0)),
            pl.BlockSpec((1, C), lambda b, t: (0, 0)),
        ],
        out_specs=pl.BlockSpec((1, C, TN), lambda b, t: (b, 0, t)),
        out_shape=jax.ShapeDtypeStruct((B, C, N), jnp.float32),
    )(ps, pss, uT, vmax, vmin, b1r, gamr, betr)
    return out


# speedup vs baseline: 9.2857x; 1.1879x over previous
import functools

import jax
import jax.numpy as jnp
from jax import lax
from jax.experimental import pallas as pl
from jax.experimental.pallas import tpu as pltpu
from jax.experimental.pallas import tpu_sc as plsc

B = 4
C = 64
N = 2048
K = 32
TN = 256
NT = N // TN
EPS = 1e-5
CNT = B * N * K

SC_CORES = 2
SC_SUBCORES = 16
NWORK = SC_CORES * SC_SUBCORES
ROWS_PER_W = CNT // NWORK
CH = 1024


def _knn_body(xf_ref, xt_ref, w_ref, idx_ref, u_ref, v_ref):
    b = pl.program_id(0)
    x = xf_ref[0]
    xt = xt_ref[0]
    sq = jnp.sum(x * x, axis=0, keepdims=True)
    inner = lax.dot_general(xt, x, (((0,), (0,)), ((), ())),
                            preferred_element_type=jnp.float32)
    score = 2.0 * inner - sq
    iota = lax.broadcasted_iota(jnp.int32, (TN, N), 1)
    cols = []
    cur = score
    m = jnp.max(cur, axis=1, keepdims=True)
    for k in range(K):
        eq = cur == m
        am = jnp.min(jnp.where(eq, iota, N), axis=1, keepdims=True)
        cols.append(am)
        cur = jnp.where(eq, -jnp.inf, cur)
        if k < K - 1:
            m = jnp.max(cur, axis=1, keepdims=True)
    idx_ref[0] = jnp.concatenate(cols, axis=1) + b * N

    w = w_ref[...]
    wa = w[:, :C]
    wb = w[:, C:]
    u_ref[0] = lax.dot_general(xt, wa - wb, (((0,), (1,)), ((), ())),
                               preferred_element_type=jnp.float32)
    v_ref[0] = lax.dot_general(xt, wb, (((0,), (1,)), ((), ())),
                               preferred_element_type=jnp.float32)


def _gather_body(idx_hbm, tab_hbm, out_hbm, idx_v, rows_v, sem):
    wid = lax.axis_index("s") * SC_CORES + lax.axis_index("c")
    for j in range(ROWS_PER_W // CH):
        base = wid * ROWS_PER_W + j * CH
        pltpu.sync_copy(idx_hbm.at[pl.ds(base, CH)], idx_v)
        pltpu.async_copy(tab_hbm.at[idx_v], rows_v, sem).wait()
        pltpu.sync_copy(rows_v, out_hbm.at[pl.ds(base, CH)])


def _reduce_body(g_ref, u_ref, b1_ref, vmax_ref, vmin_ref, ps_ref, pss_ref):
    g = g_ref[0]
    u = u_ref[0]
    b1 = b1_ref[...]
    s = g[:, 0:C]
    vmax = s
    vmin = s
    gsum = s
    gssq = s * s
    for k in range(1, K):
        s = g[:, k * C:(k + 1) * C]
        vmax = jnp.maximum(vmax, s)
        vmin = jnp.minimum(vmin, s)
        gsum = gsum + s
        gssq = gssq + s * s
    vmax_ref[0] = vmax
    vmin_ref[0] = vmin
    ub = u + b1
    ps = float(K) * ub + gsum
    pss = float(K) * ub * ub + 2.0 * ub * gsum + gssq
    ps_ref[0] = jnp.sum(ps, axis=0, keepdims=True)
    pss_ref[0] = jnp.sum(pss, axis=0, keepdims=True)


def _out_body(ps_ref, pss_ref, u_ref, vmax_ref, vmin_ref, b1_ref,
              gam_ref, bet_ref, o_ref):
    mean = jnp.sum(ps_ref[:, 0, :], axis=0, keepdims=True) / CNT
    ex2 = jnp.sum(pss_ref[:, 0, :], axis=0, keepdims=True) / CNT
    var = ex2 - mean * mean
    gam = gam_ref[...]
    bet = bet_ref[...]
    b1 = b1_ref[...]
    scale = gam * lax.rsqrt(var + EPS)
    sel = jnp.where(gam >= 0.0, vmax_ref[0], vmin_ref[0])
    h = (u_ref[0] + b1 + sel - mean) * scale + bet
    o_ref[0] = jnp.maximum(h, 0.0).T


def _gather_call():
    return functools.partial(
        pl.kernel,
        mesh=plsc.VectorSubcoreMesh(
            core_axis_name="c", subcore_axis_name="s", num_cores=SC_CORES
        ),
        out_type=jax.ShapeDtypeStruct((CNT, C), jnp.float32),
        compiler_params=pltpu.CompilerParams(use_tc_tiling_on_sc=False),
        scratch_types=[
            pltpu.VMEM((CH,), jnp.int32),
            pltpu.VMEM((CH, C), jnp.float32),
            pltpu.SemaphoreType.DMA,
        ],
    )


def kernel(points, W1, b1, gamma1, beta1):
    idx, uT, vT = pl.pallas_call(
        _knn_body,
        grid=(B, NT),
        in_specs=[
            pl.BlockSpec((1, C, N), lambda b, t: (b, 0, 0)),
            pl.BlockSpec((1, C, TN), lambda b, t: (b, 0, t)),
            pl.BlockSpec((C, 2 * C), lambda b, t: (0, 0)),
        ],
        out_specs=[
            pl.BlockSpec((1, TN, K), lambda b, t: (b, t, 0)),
            pl.BlockSpec((1, TN, C), lambda b, t: (b, t, 0)),
            pl.BlockSpec((1, TN, C), lambda b, t: (b, t, 0)),
        ],
        out_shape=[
            jax.ShapeDtypeStruct((B, N, K), jnp.int32),
            jax.ShapeDtypeStruct((B, N, C), jnp.float32),
            jax.ShapeDtypeStruct((B, N, C), jnp.float32),
        ],
    )(points, points, W1)

    idx_flat = idx.reshape(CNT)
    tab = vT.reshape(B * N, C)
    g = _gather_call()(_gather_body)(idx_flat, tab)
    g2 = g.reshape(B, N, K * C)

    b1r = b1.reshape(1, C)
    gamr = gamma1.reshape(1, C)
    betr = beta1.reshape(1, C)

    vmax, vmin, ps, pss = pl.pallas_call(
        _reduce_body,
        grid=(B, NT),
        in_specs=[
            pl.BlockSpec((1, TN, K * C), lambda b, t: (b, t, 0)),
            pl.BlockSpec((1, TN, C), lambda b, t: (b, t, 0)),
            pl.BlockSpec((1, C), lambda b, t: (0, 0)),
        ],
        out_specs=[
            pl.BlockSpec((1, TN, C), lambda b, t: (b, t, 0)),
            pl.BlockSpec((1, TN, C), lambda b, t: (b, t, 0)),
            pl.BlockSpec((1, 1, C), lambda b, t: (b * NT + t, 0, 0)),
            pl.BlockSpec((1, 1, C), lambda b, t: (b * NT + t, 0, 0)),
        ],
        out_shape=[
            jax.ShapeDtypeStruct((B, N, C), jnp.float32),
            jax.ShapeDtypeStruct((B, N, C), jnp.float32),
            jax.ShapeDtypeStruct((B * NT, 1, C), jnp.float32),
            jax.ShapeDtypeStruct((B * NT, 1, C), jnp.float32),
        ],
    )(g2, uT, b1r)

    out = pl.pallas_call(
        _out_body,
        grid=(B, NT),
        in_specs=[
            pl.BlockSpec((B * NT, 1, C), lambda b, t: (0, 0, 0)),
            pl.BlockSpec((B * NT, 1, C), lambda b, t: (0, 0, 0)),
            pl.BlockSpec((1, TN, C), lambda b, t: (b, t, 0)),
            pl.BlockSpec((1, TN, C), lambda b, t: (b, t, 0)),
            pl.BlockSpec((1, TN, C), lambda b, t: (b, t, 0)),
            pl.BlockSpec((1, C), lambda b, t: (0, 0)),
            pl.BlockSpec((1, C), lambda b, t: (0, 0)),
            pl.BlockSpec((1, C), lambda b, t: (0, 0)),
        ],
        out_specs=pl.BlockSpec((1, C, TN), lambda b, t: (b, 0, t)),
        out_shape=jax.ShapeDtypeStruct((B, C, N), jnp.float32),
    )(ps, pss, uT, vmax, vmin, b1r, gamr, betr)
    return out


# speedup vs baseline: 9.4036x; 1.0127x over previous
import functools

import jax
import jax.numpy as jnp
from jax import lax
from jax.experimental import pallas as pl
from jax.experimental.pallas import tpu as pltpu
from jax.experimental.pallas import tpu_sc as plsc

B = 4
C = 64
N = 2048
K = 32
TN = 256
NT = N // TN
EPS = 1e-5
CNT = B * N * K

SC_CORES = 2
SC_SUBCORES = 16
NWORK = SC_CORES * SC_SUBCORES
ROWS_PER_W = N * K // NWORK
CH = 1024


def _knn_body(xf_ref, xt_ref, w_ref, idx_ref, u_ref, v_ref):
    x = xf_ref[...]
    xt = xt_ref[...]
    sq = jnp.sum(x * x, axis=0, keepdims=True)
    inner = lax.dot_general(xt, x, (((0,), (0,)), ((), ())),
                            preferred_element_type=jnp.float32)
    score = 2.0 * inner - sq
    iota = lax.broadcasted_iota(jnp.int32, (TN, N), 1)
    cols = []
    cur = score
    m = jnp.max(cur, axis=1, keepdims=True)
    for k in range(K):
        eq = cur == m
        am = jnp.min(jnp.where(eq, iota, N), axis=1, keepdims=True)
        cols.append(am)
        cur = jnp.where(eq, -jnp.inf, cur)
        if k < K - 1:
            m = jnp.max(cur, axis=1, keepdims=True)
    idx_ref[...] = jnp.concatenate(cols, axis=1)

    w = w_ref[...]
    wa = w[:, :C]
    wb = w[:, C:]
    u_ref[...] = lax.dot_general(xt, wa - wb, (((0,), (1,)), ((), ())),
                                 preferred_element_type=jnp.float32)
    v_ref[...] = lax.dot_general(xt, wb, (((0,), (1,)), ((), ())),
                                 preferred_element_type=jnp.float32)


def _gather_body(idx_hbm, tab_hbm, out_hbm, idx_v, rows_v, sem):
    wid = lax.axis_index("s") * SC_CORES + lax.axis_index("c")
    for j in range(ROWS_PER_W // CH):
        base = wid * ROWS_PER_W + j * CH
        pltpu.sync_copy(idx_hbm.at[pl.ds(base, CH)], idx_v)
        pltpu.async_copy(tab_hbm.at[idx_v], rows_v, sem).wait()
        pltpu.sync_copy(rows_v, out_hbm.at[pl.ds(base, CH)])


def _reduce_body(g_ref, u_ref, b1_ref, vmax_ref, vmin_ref, ps_ref, pss_ref):
    g = g_ref[...]
    u = u_ref[...]
    b1 = b1_ref[...]
    s = g[:, 0:C]
    vmax = s
    vmin = s
    gsum = s
    gssq = s * s
    for k in range(1, K):
        s = g[:, k * C:(k + 1) * C]
        vmax = jnp.maximum(vmax, s)
        vmin = jnp.minimum(vmin, s)
        gsum = gsum + s
        gssq = gssq + s * s
    vmax_ref[...] = vmax
    vmin_ref[...] = vmin
    ub = u + b1
    ps = float(K) * ub + gsum
    pss = float(K) * ub * ub + 2.0 * ub * gsum + gssq
    ps_ref[0] = jnp.sum(ps, axis=0, keepdims=True)
    pss_ref[0] = jnp.sum(pss, axis=0, keepdims=True)


def _out_body(ps_ref, pss_ref, u_ref, vmax_ref, vmin_ref, b1_ref,
              gam_ref, bet_ref, o_ref):
    mean = jnp.sum(ps_ref[:, 0, :], axis=0, keepdims=True) / CNT
    ex2 = jnp.sum(pss_ref[:, 0, :], axis=0, keepdims=True) / CNT
    var = ex2 - mean * mean
    gam = gam_ref[...]
    bet = bet_ref[...]
    b1 = b1_ref[...]
    scale = gam * lax.rsqrt(var + EPS)
    sel = jnp.where(gam >= 0.0, vmax_ref[...], vmin_ref[...])
    h = (u_ref[...] + b1 + sel - mean) * scale + bet
    o_ref[...] = jnp.maximum(h, 0.0).T


def _gather_call():
    return functools.partial(
        pl.kernel,
        mesh=plsc.VectorSubcoreMesh(
            core_axis_name="c", subcore_axis_name="s", num_cores=SC_CORES
        ),
        out_type=jax.ShapeDtypeStruct((N * K, C), jnp.float32),
        compiler_params=pltpu.CompilerParams(use_tc_tiling_on_sc=False),
        scratch_types=[
            pltpu.VMEM((CH,), jnp.int32),
            pltpu.VMEM((CH, C), jnp.float32),
            pltpu.SemaphoreType.DMA,
        ],
    )


_knn_call = None
_reduce_call = None
_out_call = None


def _build_calls():
    global _knn_call, _reduce_call, _out_call
    if _knn_call is not None:
        return
    _knn_call = pl.pallas_call(
        _knn_body,
        grid=(NT,),
        in_specs=[
            pl.BlockSpec((C, N), lambda t: (0, 0)),
            pl.BlockSpec((C, TN), lambda t: (0, t)),
            pl.BlockSpec((C, 2 * C), lambda t: (0, 0)),
        ],
        out_specs=[
            pl.BlockSpec((TN, K), lambda t: (t, 0)),
            pl.BlockSpec((TN, C), lambda t: (t, 0)),
            pl.BlockSpec((TN, C), lambda t: (t, 0)),
        ],
        out_shape=[
            jax.ShapeDtypeStruct((N, K), jnp.int32),
            jax.ShapeDtypeStruct((N, C), jnp.float32),
            jax.ShapeDtypeStruct((N, C), jnp.float32),
        ],
    )
    _reduce_call = pl.pallas_call(
        _reduce_body,
        grid=(NT,),
        in_specs=[
            pl.BlockSpec((TN, K * C), lambda t: (t, 0)),
            pl.BlockSpec((TN, C), lambda t: (t, 0)),
            pl.BlockSpec((1, C), lambda t: (0, 0)),
        ],
        out_specs=[
            pl.BlockSpec((TN, C), lambda t: (t, 0)),
            pl.BlockSpec((TN, C), lambda t: (t, 0)),
            pl.BlockSpec((1, 1, C), lambda t: (t, 0, 0)),
            pl.BlockSpec((1, 1, C), lambda t: (t, 0, 0)),
        ],
        out_shape=[
            jax.ShapeDtypeStruct((N, C), jnp.float32),
            jax.ShapeDtypeStruct((N, C), jnp.float32),
            jax.ShapeDtypeStruct((NT, 1, C), jnp.float32),
            jax.ShapeDtypeStruct((NT, 1, C), jnp.float32),
        ],
    )
    _out_call = pl.pallas_call(
        _out_body,
        grid=(NT,),
        in_specs=[
            pl.BlockSpec((B * NT, 1, C), lambda t: (0, 0, 0)),
            pl.BlockSpec((B * NT, 1, C), lambda t: (0, 0, 0)),
            pl.BlockSpec((TN, C), lambda t: (t, 0)),
            pl.BlockSpec((TN, C), lambda t: (t, 0)),
            pl.BlockSpec((TN, C), lambda t: (t, 0)),
            pl.BlockSpec((1, C), lambda t: (0, 0)),
            pl.BlockSpec((1, C), lambda t: (0, 0)),
            pl.BlockSpec((1, C), lambda t: (0, 0)),
        ],
        out_specs=pl.BlockSpec((C, TN), lambda t: (0, t)),
        out_shape=jax.ShapeDtypeStruct((C, N), jnp.float32),
    )


def kernel(points, W1, b1, gamma1, beta1):
    _build_calls()
    b1r = b1.reshape(1, C)
    gamr = gamma1.reshape(1, C)
    betr = beta1.reshape(1, C)
    gather = _gather_call()(_gather_body)

    per_b = []
    for b in range(B):
        idx_b, uT_b, vT_b = _knn_call(points[b], points[b], W1)
        g_b = gather(idx_b.reshape(N * K), vT_b)
        per_b.append((uT_b, g_b))

    stats = []
    for b in range(B):
        uT_b, g_b = per_b[b]
        vmax_b, vmin_b, ps_b, pss_b = _reduce_call(
            g_b.reshape(N, K * C), uT_b, b1r)
        stats.append((vmax_b, vmin_b, ps_b, pss_b))

    ps_all = jnp.concatenate([s[2] for s in stats], axis=0)
    pss_all = jnp.concatenate([s[3] for s in stats], axis=0)

    outs = []
    for b in range(B):
        uT_b, _ = per_b[b]
        vmax_b, vmin_b, _, _ = stats[b]
        outs.append(_out_call(ps_all, pss_all, uT_b, vmax_b, vmin_b,
                              b1r, gamr, betr))
    return jnp.stack(outs, axis=0)
